# Initial kernel scaffold; baseline (speedup 1.0000x reference)
import jax
import jax.numpy as jnp
from jax.experimental import pallas as pl


def kernel(hidden_states, key_pool, value_pool, fc_w, fc_b):
    raise NotImplementedError("write your pallas kernel here")


# speedup vs baseline: 9.6719x; 9.6719x over previous
import functools

import jax
import jax.numpy as jnp
from jax import lax
from jax.experimental import pallas as pl
from jax.experimental.pallas import tpu as pltpu

TOPK = 8
POOL = 64
TM = 512


def _vw_body(value_pool_ref, w2_ref, vw_ref):
    vw_ref[...] = lax.dot_general(
        value_pool_ref[...], w2_ref[...],
        (((1,), (1,)), ((), ())),
        preferred_element_type=jnp.float32,
    )


def _build_vw(value_pool, fc_w, emb_dim):
    return pl.pallas_call(
        _vw_body,
        grid=(TOPK,),
        in_specs=[
            pl.BlockSpec((POOL, emb_dim), lambda j: (0, 0)),
            pl.BlockSpec((emb_dim, emb_dim), lambda j: (0, j + 1)),
        ],
        out_specs=pl.BlockSpec((POOL, emb_dim), lambda j: (j, 0)),
        out_shape=jax.ShapeDtypeStruct((TOPK * POOL, emb_dim), jnp.float32),
    )(value_pool, fc_w)


def _main_body(h_ref, kp_ref, fcw_ref, vw_ref, b_ref,
               out_ref, idx_ref, sim_ref, sim_acc):
    i = pl.program_id(0)
    n_steps = pl.num_programs(0)

    h = h_ref[...]
    kp = kp_ref[...]

    hn = h / jnp.maximum(jnp.sqrt(jnp.sum(h * h, axis=1, keepdims=True)), 1e-12)
    kn = kp / jnp.maximum(jnp.sqrt(jnp.sum(kp * kp, axis=1, keepdims=True)), 1e-12)

    w = lax.dot_general(hn, kn, (((1,), (1,)), ((), ())),
                        preferred_element_type=jnp.float32)

    iota = lax.broadcasted_iota(jnp.int32, (TM, POOL), 1)
    topsum = jnp.zeros((TM, 1), jnp.float32)
    oh_cols = []
    for j in range(TOPK):
        m = jnp.max(w, axis=1, keepdims=True)
        sel = w == m
        idx_j = jnp.min(jnp.where(sel, iota, POOL), axis=1)
        oh = (iota == idx_j[:, None])
        w = jnp.where(oh, -jnp.inf, w)
        topsum = topsum + m
        oh_cols.append(oh.astype(jnp.float32))
        idx_ref[j, :] = idx_j
    p_oh = jnp.concatenate(oh_cols, axis=1)

    pool_contrib = lax.dot_general(
        p_oh, vw_ref[...], (((1,), (0,)), ((), ())),
        preferred_element_type=jnp.float32,
        precision=lax.Precision.HIGHEST)

    dense = lax.dot_general(h, fcw_ref[...], (((1,), (1,)), ((), ())),
                            preferred_element_type=jnp.float32)

    out_ref[...] = dense + pool_contrib + b_ref[...]

    @pl.when(i == 0)
    def _():
        sim_acc[0] = 0.0
    sim_acc[0] += jnp.sum(topsum)

    @pl.when(i == n_steps - 1)
    def _():
        sim_ref[...] = jnp.reshape(sim_acc[0], (1, 1))


def kernel(hidden_states, key_pool, value_pool, fc_w, fc_b):
    B, T, D = hidden_states.shape
    O = fc_w.shape[0]
    N = B * T
    h2 = hidden_states.reshape(N, D)
    vw = _build_vw(value_pool, fc_w, D)

    grid = (N // TM,)
    out2, _idx, sim = pl.pallas_call(
        _main_body,
        grid=grid,
        in_specs=[
            pl.BlockSpec((TM, D), lambda i: (i, 0)),
            pl.BlockSpec((POOL, D), lambda i: (0, 0)),
            pl.BlockSpec((O, D), lambda i: (0, 0)),
            pl.BlockSpec((TOPK * POOL, O), lambda i: (0, 0)),
            pl.BlockSpec((1, O), lambda i: (0, 0)),
        ],
        out_specs=[
            pl.BlockSpec((TM, O), lambda i: (i, 0)),
            pl.BlockSpec((TOPK, TM), lambda i: (0, i)),
            pl.BlockSpec((1, 1), lambda i: (0, 0)),
        ],
        out_shape=[
            jax.ShapeDtypeStruct((N, O), jnp.float32),
            jax.ShapeDtypeStruct((TOPK, N), jnp.int32),
            jax.ShapeDtypeStruct((1, 1), jnp.float32),
        ],
        scratch_shapes=[pltpu.SMEM((1,), jnp.float32)],
    )(h2, key_pool, fc_w, vw, fc_b.reshape(1, O))

    out = out2.reshape(B, T, O)
    sim_loss = sim[0, 0] / B
    return (out, sim_loss)
